# Initial kernel scaffold; baseline (speedup 1.0000x reference)
import jax
import jax.numpy as jnp
from jax.experimental import pallas as pl


def kernel(actor_ctrs, node_ctrs, node_feats):
    raise NotImplementedError("write your pallas kernel here")


# speedup vs baseline: 1.3406x; 1.3406x over previous
import functools

import jax
import jax.numpy as jnp
from jax.experimental import pallas as pl
from jax.experimental.pallas import tpu as pltpu

_NEG = -1e30


def _ctx_kernel(ax_ref, ay_ref, nxt_ref, nyt_ref, nf_ref, out_ref, *,
                dist_sq, ga, ns):
    a_total = ax_ref.shape[0]
    n_total = nxt_ref.shape[1]
    h = nf_ref.shape[1]
    n_groups = a_total // ga
    n_slabs = n_total // ns

    def group_body(g, carry):
        a0 = g * ga
        ax = ax_ref[pl.ds(a0, ga), :]
        ay = ay_ref[pl.ds(a0, ga), :]

        def slab_body(s, red):
            s0 = s * ns
            nx = nxt_ref[:, pl.ds(s0, ns)]
            ny = nyt_ref[:, pl.ds(s0, ns)]
            dx = ax - nx
            dy = ay - ny
            pen = jnp.where((dx * dx + dy * dy) <= dist_sq, 0.0, _NEG)
            nf_s = nf_ref[pl.ds(s0, ns), :]
            masked = nf_s[None, :, :] + pen[:, :, None]
            return jnp.maximum(red, jnp.max(masked, axis=1))

        red0 = jnp.full((ga, h), _NEG, jnp.float32)
        red = jax.lax.fori_loop(0, n_slabs, slab_body, red0)
        out_ref[pl.ds(a0, ga), :] = jnp.where(red > 0.5 * _NEG, red, 0.0)
        return carry

    jax.lax.fori_loop(0, n_groups, group_body, 0)


def kernel(actor_ctrs, node_ctrs, node_feats):
    B, A, _ = actor_ctrs.shape
    _, N, H = node_feats.shape
    dist_th = 6.0

    f32 = jnp.float32
    ax = actor_ctrs[..., 0:1].astype(f32)
    ay = actor_ctrs[..., 1:2].astype(f32)
    nxt = node_ctrs[..., 0].astype(f32).reshape(B, 1, N)
    nyt = node_ctrs[..., 1].astype(f32).reshape(B, 1, N)
    nf = node_feats.astype(f32)

    GA = 8
    NS = 512

    kern = functools.partial(_ctx_kernel, dist_sq=float(dist_th) ** 2,
                             ga=GA, ns=NS)
    ctx = pl.pallas_call(
        kern,
        out_shape=jax.ShapeDtypeStruct((B, A, H), jnp.float32),
        grid=(B,),
        in_specs=[
            pl.BlockSpec((None, A, 1), lambda b: (b, 0, 0)),
            pl.BlockSpec((None, A, 1), lambda b: (b, 0, 0)),
            pl.BlockSpec((None, 1, N), lambda b: (b, 0, 0)),
            pl.BlockSpec((None, 1, N), lambda b: (b, 0, 0)),
            pl.BlockSpec((None, N, H), lambda b: (b, 0, 0)),
        ],
        out_specs=pl.BlockSpec((None, A, H), lambda b: (b, 0, 0)),
        compiler_params=pltpu.CompilerParams(
            dimension_semantics=("parallel",),
            vmem_limit_bytes=64 << 20),
    )(ax, ay, nxt, nyt, nf)

    return ctx.reshape(B * A, H)
